# Initial kernel scaffold; baseline (speedup 1.0000x reference)
import jax
import jax.numpy as jnp
from jax.experimental import pallas as pl


def kernel(x, table, pe):
    raise NotImplementedError("write your pallas kernel here")


# speedup vs baseline: 1.1773x; 1.1773x over previous
import functools

import jax
import jax.numpy as jnp
from jax import lax
from jax.experimental import pallas as pl
from jax.experimental.pallas import tpu as pltpu
from jax.experimental.pallas import tpu_sc as plsc

B = 1024
L = 200
D = 128
N = B * L
NC, NS = 2, 16
NW = NC * NS
TOK_PER_W = N // NW
G = 128
NG = TOK_PER_W // G
NBUF = 5
NOUT = NG // NBUF


@functools.lru_cache(maxsize=1)
def _sc_gather_fn():
  mesh = plsc.VectorSubcoreMesh(
      core_axis_name="c", subcore_axis_name="s", num_cores=NC,
      num_subcores=NS)

  sems = [pltpu.SemaphoreType.DMA for _ in range(2 * NBUF)]

  @functools.partial(
      pl.kernel,
      out_type=jax.ShapeDtypeStruct((N, D), jnp.float32),
      mesh=mesh,
      scratch_types=[
          pltpu.VMEM((NG, G), jnp.int32),
          pltpu.VMEM((NBUF, G, D), jnp.float32),
      ] + sems,
  )
  def sc_gather(x_hbm, table_hbm, out_hbm, idx_v, rows_v, *all_sems):
    gsem = all_sems[:NBUF]
    osem = all_sems[NBUF:]
    wid = lax.axis_index("s") * NC + lax.axis_index("c")
    base = wid * TOK_PER_W

    pltpu.sync_copy(x_hbm.at[wid], idx_v)

    def gstart(j, b):
      pltpu.async_copy(table_hbm.at[idx_v.at[j]], rows_v.at[b], gsem[b])

    def gwait(b):
      pltpu.make_async_copy(
          out_hbm.at[pl.ds(0, G)], rows_v.at[b], gsem[b]).wait()

    def ostart(j, b):
      pltpu.async_copy(
          rows_v.at[b], out_hbm.at[pl.ds(base + j * G, G)], osem[b])

    def owait(b):
      pltpu.make_async_copy(
          rows_v.at[b], out_hbm.at[pl.ds(0, G)], osem[b]).wait()

    for b in range(NBUF):
      gstart(b, b)

    def body(outer, carry):
      for b in range(NBUF):
        j = outer * NBUF + b
        gwait(b)
        ostart(j, b)
      for b in range(NBUF):
        jn = (outer + 1) * NBUF + b
        owait(b)
        gstart(jn, b)
      return carry

    lax.fori_loop(0, NOUT - 1, body, 0)

    for b in range(NBUF):
      j = (NOUT - 1) * NBUF + b
      gwait(b)
      ostart(j, b)
    for b in range(NBUF):
      owait(b)

  return sc_gather


BB = 8


def _tc_post_body(x_ref, raw_ref, pe_ref, emb_ref, pad_ref, caus_ref):
  x = x_ref[...]
  pad = x == 0
  emb_ref[...] = raw_ref[...] + pe_ref[...][None]
  pad_ref[...] = pad
  row = lax.broadcasted_iota(jnp.int32, (L, L), 0)
  col = lax.broadcasted_iota(jnp.int32, (L, L), 1)
  tri = col > row
  caus_ref[...] = pad[:, None, :] | tri[None]


def _tc_post(x, raw, pe2):
  grid = (B // BB,)
  return pl.pallas_call(
      _tc_post_body,
      grid=grid,
      in_specs=[
          pl.BlockSpec((BB, L), lambda i: (i, 0)),
          pl.BlockSpec((BB, L, D), lambda i: (i, 0, 0)),
          pl.BlockSpec((L, D), lambda i: (0, 0)),
      ],
      out_specs=[
          pl.BlockSpec((BB, L, D), lambda i: (i, 0, 0)),
          pl.BlockSpec((BB, L), lambda i: (i, 0)),
          pl.BlockSpec((BB, L, L), lambda i: (i, 0, 0)),
      ],
      out_shape=[
          jax.ShapeDtypeStruct((B, L, D), jnp.float32),
          jax.ShapeDtypeStruct((B, L), jnp.bool_),
          jax.ShapeDtypeStruct((B, L, L), jnp.bool_),
      ],
  )(x, raw, pe2)


@jax.jit
def kernel(x, table, pe):
  xflat = x.reshape(NW, NG, G)
  raw = _sc_gather_fn()(xflat, table)
  pe2 = pe[0, :L]
  emb, pad, caus = _tc_post(x, raw.reshape(B, L, D), pe2)
  return emb, pad[:, None, None, :], caus[:, None]


# speedup vs baseline: 1.2432x; 1.0560x over previous
import functools

import jax
import jax.numpy as jnp
from jax import lax
from jax.experimental import pallas as pl
from jax.experimental.pallas import tpu as pltpu
from jax.experimental.pallas import tpu_sc as plsc

B = 1024
L = 200
D = 128
NC, NS = 2, 16
NW = NC * NS
SEQ_PER_W = B // NW
NBUF = 3
NROUND = SEQ_PER_W // NBUF


@functools.lru_cache(maxsize=1)
def _sc_embed_fn():
  mesh = plsc.VectorSubcoreMesh(
      core_axis_name="c", subcore_axis_name="s", num_cores=NC,
      num_subcores=NS)

  sems = [pltpu.SemaphoreType.DMA for _ in range(2 * NBUF)]

  @functools.partial(
      pl.kernel,
      out_type=jax.ShapeDtypeStruct((B, L, D), jnp.float32),
      mesh=mesh,
      scratch_types=[
          pltpu.VMEM((SEQ_PER_W, L), jnp.int32),
          pltpu.VMEM((L, D), jnp.float32),
          pltpu.VMEM((NBUF, L, D), jnp.float32),
      ] + sems,
  )
  def sc_embed(x_hbm, table_hbm, pe_hbm, out_hbm, idx_v, pe_v, buf_v,
               *all_sems):
    gsem = all_sems[:NBUF]
    osem = all_sems[NBUF:]
    wid = lax.axis_index("s") * NC + lax.axis_index("c")
    base = wid * SEQ_PER_W

    pltpu.sync_copy(x_hbm.at[pl.ds(base, SEQ_PER_W)], idx_v)
    pltpu.sync_copy(pe_hbm, pe_v)

    def gstart(j, b):
      pltpu.async_copy(table_hbm.at[idx_v.at[j, pl.ds(0, 128)]],
                       buf_v.at[b, pl.ds(0, 128)], gsem[b])
      pltpu.async_copy(table_hbm.at[idx_v.at[j, pl.ds(128, 72)]],
                       buf_v.at[b, pl.ds(128, 72)], gsem[b])

    def gwait(b):
      pltpu.make_async_copy(
          table_hbm.at[pl.ds(0, L)], buf_v.at[b], gsem[b]).wait()

    def ostart(j, b):
      pltpu.async_copy(buf_v.at[b], out_hbm.at[base + j], osem[b])

    def owait(b):
      pltpu.make_async_copy(buf_v.at[b], out_hbm.at[0], osem[b]).wait()

    def add_pe(b):
      def row(r, carry):
        for c in range(D // 16):
          sl = pl.ds(c * 16, 16)
          buf_v[b, r, sl] = buf_v[b, r, sl] + pe_v[r, sl]
        return carry
      lax.fori_loop(0, L, row, 0)

    for b in range(NBUF):
      gstart(b, b)

    def round_body(r, carry):
      for b in range(NBUF):
        j = r * NBUF + b
        gwait(b)
        add_pe(b)
        ostart(j, b)
      for b in range(NBUF):
        owait(b)
        gstart(r * NBUF + NBUF + b, b)
      return carry

    lax.fori_loop(0, NROUND - 1, round_body, 0)

    tail = (NROUND - 1) * NBUF
    rem = SEQ_PER_W - NROUND * NBUF
    for b in range(NBUF):
      gwait(b)
      add_pe(b)
      ostart(tail + b, b)
    for b in range(rem):
      owait(b)
      gstart(NROUND * NBUF + b, b)
    for b in range(rem):
      gwait(b)
      add_pe(b)
      ostart(NROUND * NBUF + b, b)
    for b in range(rem, NBUF):
      owait(b)
    for b in range(rem):
      owait(b)

  return sc_embed


BB = 8


def _tc_masks_body(x_ref, pad_ref, caus_ref):
  x = x_ref[...]
  pad = x == 0
  pad_ref[...] = pad[:, None, None, :]
  row = lax.broadcasted_iota(jnp.int32, (L, L), 0)
  col = lax.broadcasted_iota(jnp.int32, (L, L), 1)
  tri = col > row
  caus_ref[...] = pad[:, None, None, :] | tri[None, None]


def _tc_masks(x):
  return pl.pallas_call(
      _tc_masks_body,
      grid=(B // BB,),
      in_specs=[
          pl.BlockSpec((BB, L), lambda i: (i, 0)),
      ],
      out_specs=[
          pl.BlockSpec((BB, 1, 1, L), lambda i: (i, 0, 0, 0)),
          pl.BlockSpec((BB, 1, L, L), lambda i: (i, 0, 0, 0)),
      ],
      out_shape=[
          jax.ShapeDtypeStruct((B, 1, 1, L), jnp.bool_),
          jax.ShapeDtypeStruct((B, 1, L, L), jnp.bool_),
      ],
  )(x)


@jax.jit
def kernel(x, table, pe):
  pe2 = pe[0, :L]
  emb = _sc_embed_fn()(x, table, pe2)
  pad, caus = _tc_masks(x)
  return emb, pad, caus


# speedup vs baseline: 5.2335x; 4.2097x over previous
import functools

import jax
import jax.numpy as jnp
from jax import lax
from jax.experimental import pallas as pl
from jax.experimental.pallas import tpu as pltpu
from jax.experimental.pallas import tpu_sc as plsc

B = 1024
L = 200
D = 128
NC, NS = 2, 16
NW = NC * NS
SEQ_PER_W = B // NW
NBUF = 3
NROUND = SEQ_PER_W // NBUF


@functools.lru_cache(maxsize=1)
def _sc_embed_fn():
  mesh = plsc.VectorSubcoreMesh(
      core_axis_name="c", subcore_axis_name="s", num_cores=NC,
      num_subcores=NS)

  sems = [pltpu.SemaphoreType.DMA for _ in range(2 * NBUF)]

  @functools.partial(
      pl.kernel,
      out_type=jax.ShapeDtypeStruct((B, L, D), jnp.float32),
      mesh=mesh,
      compiler_params=pltpu.CompilerParams(use_tc_tiling_on_sc=True),
      scratch_types=[
          pltpu.VMEM((SEQ_PER_W, L), jnp.int32),
          pltpu.VMEM((L, D), jnp.float32),
          pltpu.VMEM((NBUF, L, D), jnp.float32),
      ] + sems,
  )
  def sc_embed(x_hbm, table_hbm, pe_hbm, out_hbm, idx_v, pe_v, buf_v,
               *all_sems):
    gsem = all_sems[:NBUF]
    osem = all_sems[NBUF:]
    wid = lax.axis_index("s") * NC + lax.axis_index("c")
    base = wid * SEQ_PER_W

    pltpu.sync_copy(x_hbm.at[pl.ds(base, SEQ_PER_W)], idx_v)
    pltpu.sync_copy(pe_hbm, pe_v)

    def gstart(j, b):
      pltpu.async_copy(table_hbm.at[idx_v.at[j, pl.ds(0, 128)]],
                       buf_v.at[b, pl.ds(0, 128)], gsem[b])
      pltpu.async_copy(table_hbm.at[idx_v.at[j, pl.ds(128, 72)]],
                       buf_v.at[b, pl.ds(128, 72)], gsem[b])

    def gwait(b):
      pltpu.make_async_copy(
          table_hbm.at[pl.ds(0, L)], buf_v.at[b], gsem[b]).wait()

    def ostart(j, b):
      pltpu.async_copy(buf_v.at[b], out_hbm.at[base + j], osem[b])

    def owait(b):
      pltpu.make_async_copy(buf_v.at[b], out_hbm.at[0], osem[b]).wait()

    def add_pe(b):
      def row(r, carry):
        for c in range(D // 16):
          sl = pl.ds(c * 16, 16)
          buf_v[b, r, sl] = buf_v[b, r, sl] + pe_v[r, sl]
        return carry
      lax.fori_loop(0, L, row, 0)

    for b in range(NBUF):
      gstart(b, b)

    def round_body(r, carry):
      for b in range(NBUF):
        j = r * NBUF + b
        gwait(b)
        add_pe(b)
        ostart(j, b)
      for b in range(NBUF):
        owait(b)
        gstart(r * NBUF + NBUF + b, b)
      return carry

    lax.fori_loop(0, NROUND - 1, round_body, 0)

    tail = (NROUND - 1) * NBUF
    rem = SEQ_PER_W - NROUND * NBUF
    for b in range(NBUF):
      gwait(b)
      add_pe(b)
      ostart(tail + b, b)
    for b in range(rem):
      owait(b)
      gstart(NROUND * NBUF + b, b)
    for b in range(rem):
      gwait(b)
      add_pe(b)
      ostart(NROUND * NBUF + b, b)
    for b in range(rem, NBUF):
      owait(b)
    for b in range(rem):
      owait(b)

  return sc_embed


BB = 128


def _tc_masks_body(xt_ref, pad_ref, caus_ref):
  xt = xt_ref[...]
  pad = (xt == 0).astype(jnp.int8)
  pad_ref[...] = pad[None, None]
  row = lax.broadcasted_iota(jnp.int32, (L, L), 0)
  col = lax.broadcasted_iota(jnp.int32, (L, L), 1)
  tri = (col > row).astype(jnp.int8)
  caus_ref[...] = (pad[None, :, :] | tri[:, :, None])[None]


def _tc_masks(xt):
  return pl.pallas_call(
      _tc_masks_body,
      grid=(B // BB,),
      in_specs=[
          pl.BlockSpec((L, BB), lambda i: (0, i)),
      ],
      out_specs=[
          pl.BlockSpec((1, 1, L, BB), lambda i: (0, 0, 0, i)),
          pl.BlockSpec((1, L, L, BB), lambda i: (0, 0, 0, i)),
      ],
      out_shape=[
          jax.ShapeDtypeStruct((1, 1, L, B), jnp.int8),
          jax.ShapeDtypeStruct((1, L, L, B), jnp.int8),
      ],
  )(xt)


@jax.jit
def kernel(x, table, pe):
  pe2 = pe[0, :L]
  emb = _sc_embed_fn()(x, table, pe2)
  pad8, caus8 = _tc_masks(x.T)
  pad = lax.transpose(pad8, (3, 0, 1, 2)).astype(jnp.bool_)
  caus = lax.transpose(caus8, (3, 0, 1, 2)).astype(jnp.bool_)
  return emb, pad, caus


# speedup vs baseline: 5.5765x; 1.0655x over previous
import functools

import jax
import jax.numpy as jnp
from jax import lax
from jax.experimental import pallas as pl
from jax.experimental.pallas import tpu as pltpu
from jax.experimental.pallas import tpu_sc as plsc

B = 1024
L = 200
D = 128
NC, NS = 2, 16
NW = NC * NS
SEQ_PER_W = B // NW
NBUF = 3
NROUND = SEQ_PER_W // NBUF


@functools.lru_cache(maxsize=1)
def _sc_embed_fn():
  mesh = plsc.VectorSubcoreMesh(
      core_axis_name="c", subcore_axis_name="s", num_cores=NC,
      num_subcores=NS)

  sems = [pltpu.SemaphoreType.DMA for _ in range(2 * NBUF)]

  @functools.partial(
      pl.kernel,
      out_type=jax.ShapeDtypeStruct((B, L, D), jnp.float32),
      mesh=mesh,
      compiler_params=pltpu.CompilerParams(use_tc_tiling_on_sc=True),
      scratch_types=[
          pltpu.VMEM((SEQ_PER_W, L), jnp.int32),
          pltpu.VMEM((L, D), jnp.float32),
          pltpu.VMEM((NBUF, L, D), jnp.float32),
      ] + sems,
  )
  def sc_embed(x_hbm, table_hbm, pe_hbm, out_hbm, idx_v, pe_v, buf_v,
               *all_sems):
    gsem = all_sems[:NBUF]
    osem = all_sems[NBUF:]
    wid = lax.axis_index("s") * NC + lax.axis_index("c")
    base = wid * SEQ_PER_W

    pltpu.sync_copy(x_hbm.at[pl.ds(base, SEQ_PER_W)], idx_v)
    pltpu.sync_copy(pe_hbm, pe_v)

    def gstart(j, b):
      pltpu.async_copy(table_hbm.at[idx_v.at[j, pl.ds(0, 128)]],
                       buf_v.at[b, pl.ds(0, 128)], gsem[b])
      pltpu.async_copy(table_hbm.at[idx_v.at[j, pl.ds(128, 72)]],
                       buf_v.at[b, pl.ds(128, 72)], gsem[b])

    def gwait(b):
      pltpu.make_async_copy(
          table_hbm.at[pl.ds(0, L)], buf_v.at[b], gsem[b]).wait()

    def ostart(j, b):
      pltpu.async_copy(buf_v.at[b], out_hbm.at[base + j], osem[b])

    def owait(b):
      pltpu.make_async_copy(buf_v.at[b], out_hbm.at[0], osem[b]).wait()

    def add_pe(b):
      def rows(r2, carry):
        for u in range(2):
          for c in range(D // 16):
            sl = pl.ds(c * 16, 16)
            buf_v[b, r2 * 2 + u, sl] = (
                buf_v[b, r2 * 2 + u, sl] + pe_v[r2 * 2 + u, sl])
        return carry
      lax.fori_loop(0, L // 2, rows, 0)

    def step(j, b, first, last):
      gwait(b)
      add_pe(b)
      ostart(j, b)
      if not first:
        owait((b + 2) % NBUF)
      if not last:
        gstart(j + 2, (b + 2) % NBUF)

    gstart(0, 0)
    gstart(1, 1)
    step(0, 0, True, False)
    step(1, 1, False, False)

    def round_body(r, carry):
      for i in range(NBUF):
        step(2 + NBUF * r + i, (2 + i) % NBUF, False, False)
      return carry

    lax.fori_loop(0, (SEQ_PER_W - 2) // NBUF - 1, round_body, 0)

    for j in range(SEQ_PER_W - NBUF, SEQ_PER_W):
      step(j, j % NBUF, False, j + 2 >= SEQ_PER_W)
    owait((SEQ_PER_W - 1) % NBUF)

  return sc_embed


BB = 128


def _tc_masks_body(xt_ref, pad_ref, caus_ref):
  xt = xt_ref[...]
  pad = (xt == 0).astype(jnp.int8)
  pad_ref[...] = pad[None, None]
  row = lax.broadcasted_iota(jnp.int32, (L, L), 0)
  col = lax.broadcasted_iota(jnp.int32, (L, L), 1)
  tri = (col > row).astype(jnp.int8)
  caus_ref[...] = (pad[None, :, :] | tri[:, :, None])[None]


def _tc_masks(xt):
  return pl.pallas_call(
      _tc_masks_body,
      grid=(B // BB,),
      in_specs=[
          pl.BlockSpec((L, BB), lambda i: (0, i)),
      ],
      out_specs=[
          pl.BlockSpec((1, 1, L, BB), lambda i: (0, 0, 0, i)),
          pl.BlockSpec((1, L, L, BB), lambda i: (0, 0, 0, i)),
      ],
      out_shape=[
          jax.ShapeDtypeStruct((1, 1, L, B), jnp.int8),
          jax.ShapeDtypeStruct((1, L, L, B), jnp.int8),
      ],
  )(xt)


@jax.jit
def kernel(x, table, pe):
  pe2 = pe[0, :L]
  emb = _sc_embed_fn()(x, table, pe2)
  pad8, caus8 = _tc_masks(x.T)
  pad = lax.transpose(pad8, (3, 0, 1, 2)).astype(jnp.bool_)
  caus = lax.transpose(caus8, (3, 0, 1, 2)).astype(jnp.bool_)
  return emb, pad, caus
